# Initial kernel scaffold; baseline (speedup 1.0000x reference)
import jax
import jax.numpy as jnp
from jax.experimental import pallas as pl


def kernel(x, edge_index, h0, lamda, alpha, return_loss, cums, fc_w, attn_l, attn_r, gat_bias, w2, cls_w, cls_b):
    raise NotImplementedError("write your pallas kernel here")


# speedup vs baseline: 38.9711x; 38.9711x over previous
import functools
import math

import jax
import jax.numpy as jnp
from jax import lax
from jax.experimental import pallas as pl
from jax.experimental.pallas import tpu as pltpu
from jax.experimental.pallas import tpu_sc as plsc

N = 10000
E = 320000
D = 128
H = 8
HD = 16
L = 4

NC = 2
NS = 16
NW = NC * NS
EPW = E // NW
CH = 80
NCHUNK = EPW // CH
RPS = N // NS
ZR = 25

TSW = D + HD
TDW = HD

_f32 = jnp.float32


def _prep_body(x_ref, w_ref, ms_ref, mt_ref, ts_ref, td_ref):
    f = jnp.dot(x_ref[...], w_ref[...], preferred_element_type=_f32)
    ts_ref[...] = jnp.dot(f, ms_ref[...], preferred_element_type=_f32)
    td_ref[...] = jnp.dot(f, mt_ref[...], preferred_element_type=_f32)


def _prep_call(x, w, ms, mt):
    br = 1000
    grid = N // br
    return pl.pallas_call(
        _prep_body,
        grid=(grid,),
        in_specs=[
            pl.BlockSpec((br, D), lambda i: (i, 0)),
            pl.BlockSpec((D, D), lambda i: (0, 0)),
            pl.BlockSpec((D, TSW), lambda i: (0, 0)),
            pl.BlockSpec((D, TDW), lambda i: (0, 0)),
        ],
        out_specs=[
            pl.BlockSpec((br, TSW), lambda i: (i, 0)),
            pl.BlockSpec((br, TDW), lambda i: (i, 0)),
        ],
        out_shape=[
            jax.ShapeDtypeStruct((N, TSW), _f32),
            jax.ShapeDtypeStruct((N, TDW), _f32),
        ],
    )(x, w, ms, mt)


def _edge_body(src_h, dst_h, ts_h, td_h, acc_h, den_h,
               src_v, dst_v, ts_v, td_v, wf_v, ee_v, zacc_v, zden_v,
               acc_s, den_s, sem):
    c = lax.axis_index("c")
    s = lax.axis_index("s")
    wid = c * NS + s
    zero = jnp.zeros((16,), _f32)

    def zfill_a(i, _):
        for k in range(D // 16):
            zacc_v[i, pl.ds(16 * k, 16)] = zero
        return 0
    lax.fori_loop(0, ZR, zfill_a, 0)

    def zfill_d(i, _):
        zden_v[i, :] = zero
        return 0
    lax.fori_loop(0, 5 * ZR, zfill_d, 0)
    row0 = s * RPS
    for j in range(RPS // ZR):
        pltpu.sync_copy(zacc_v, acc_s.at[pl.ds(row0 + j * ZR, ZR)])
    for j in range(RPS // (5 * ZR)):
        pltpu.sync_copy(zden_v, den_s.at[pl.ds(row0 + j * 5 * ZR, 5 * ZR)])
    plsc.subcore_barrier()

    ebase = wid * EPW

    def chunk_body(j, _):
        off = ebase + j * CH
        pltpu.sync_copy(src_h.at[pl.ds(off, CH)], src_v)
        pltpu.sync_copy(dst_h.at[pl.ds(off, CH)], dst_v)
        pltpu.async_copy(ts_h.at[src_v], ts_v, sem).wait()
        pltpu.async_copy(td_h.at[dst_v], td_v, sem).wait()

        def edge_body(i, _):
            e = ts_v[i, pl.ds(D, HD)] + td_v[i, :]
            e = jnp.maximum(e, 0.2 * e)
            e = jnp.minimum(e, 80.0)
            ee = jnp.exp(e)
            ee_v[i, :] = ee
            for m in range(H):
                wf_v[i, pl.ds(HD * m, HD)] = (
                    ee * ts_v[i, pl.ds(HD * m, HD)])
            return 0
        lax.fori_loop(0, CH, edge_body, 0)

        pltpu.sync_copy(wf_v, acc_s.at[dst_v], add=True)
        pltpu.sync_copy(ee_v, den_s.at[dst_v], add=True)
        return 0
    lax.fori_loop(0, NCHUNK, chunk_body, 0)
    plsc.subcore_barrier()

    for j in range(RPS // ZR):
        r = row0 + j * ZR
        pltpu.sync_copy(acc_s.at[pl.ds(r, ZR)], acc_h.at[c, pl.ds(r, ZR)])
    pltpu.sync_copy(den_s.at[pl.ds(row0, RPS)], den_h.at[c, pl.ds(row0, RPS)])


_edge_call = functools.partial(
    pl.kernel,
    out_type=[
        jax.ShapeDtypeStruct((NC, N, D), _f32),
        jax.ShapeDtypeStruct((NC, N, HD), _f32),
    ],
    mesh=plsc.VectorSubcoreMesh(core_axis_name="c", subcore_axis_name="s"),
    scratch_types=[
        pltpu.VMEM((CH,), jnp.int32),
        pltpu.VMEM((CH,), jnp.int32),
        pltpu.VMEM((CH, TSW), _f32),
        pltpu.VMEM((CH, TDW), _f32),
        pltpu.VMEM((CH, D), _f32),
        pltpu.VMEM((CH, HD), _f32),
        pltpu.VMEM((ZR, D), _f32),
        pltpu.VMEM((5 * ZR, HD), _f32),
        pltpu.VMEM_SHARED((N, D), _f32),
        pltpu.VMEM_SHARED((N, HD), _f32),
        pltpu.SemaphoreType.DMA,
    ],
    compiler_params=pltpu.CompilerParams(use_tc_tiling_on_sc=False),
)(_edge_body)


def _mix_body(acc_ref, den_ref, h0_ref, xp_ref, w2a_ref, w2b_ref,
              bias_ref, rmat_ref, pinv_ref, coef_ref, xn_ref):
    accs = acc_ref[0] + acc_ref[1]
    dens = den_ref[0] + den_ref[1]
    den_exp = jnp.dot(dens, rmat_ref[...], preferred_element_type=_f32)
    gx_int = jnp.where(den_exp > 0.0, accs / den_exp, 0.0)
    gx = jnp.dot(gx_int, pinv_ref[...], preferred_element_type=_f32)
    gx = gx + bias_ref[...]
    h0 = h0_ref[...]
    mm = (jnp.dot(gx, w2a_ref[...], preferred_element_type=_f32)
          + jnp.dot(h0, w2b_ref[...], preferred_element_type=_f32))
    c1 = coef_ref[0, 0]
    c2 = coef_ref[0, 1]
    c3 = coef_ref[0, 2]
    xn_ref[...] = c1 * mm + c2 * gx + c3 * h0 + xp_ref[...]


def _mix_call(accp, denp, h0, xp, w2a, w2b, bias, rmat, pinv, coefs):
    br = 1000
    grid = N // br
    return pl.pallas_call(
        _mix_body,
        grid=(grid,),
        in_specs=[
            pl.BlockSpec((NC, br, D), lambda i: (0, i, 0)),
            pl.BlockSpec((NC, br, HD), lambda i: (0, i, 0)),
            pl.BlockSpec((br, D), lambda i: (i, 0)),
            pl.BlockSpec((br, D), lambda i: (i, 0)),
            pl.BlockSpec((D, D), lambda i: (0, 0)),
            pl.BlockSpec((D, D), lambda i: (0, 0)),
            pl.BlockSpec((1, D), lambda i: (0, 0)),
            pl.BlockSpec((HD, D), lambda i: (0, 0)),
            pl.BlockSpec((D, D), lambda i: (0, 0)),
            pl.BlockSpec(memory_space=pltpu.SMEM),
        ],
        out_specs=pl.BlockSpec((br, D), lambda i: (i, 0)),
        out_shape=jax.ShapeDtypeStruct((N, D), _f32),
    )(accp, denp, h0, xp, w2a, w2b, bias, rmat, pinv, coefs)


def _cls_body(x_ref, w_ref, b_ref, logit_ref, y_ref):
    lg = jnp.dot(x_ref[...], w_ref[...], preferred_element_type=_f32)
    lg = lg + b_ref[...]
    logit_ref[...] = lg
    y_ref[...] = jax.nn.sigmoid(lg[:, 1:2] - lg[:, 0:1])


def _cls_call(x, w, b):
    br = 1000
    grid = N // br
    return pl.pallas_call(
        _cls_body,
        grid=(grid,),
        in_specs=[
            pl.BlockSpec((br, D), lambda i: (i, 0)),
            pl.BlockSpec((D, 2), lambda i: (0, 0)),
            pl.BlockSpec((1, 2), lambda i: (0, 0)),
        ],
        out_specs=[
            pl.BlockSpec((br, 2), lambda i: (i, 0)),
            pl.BlockSpec((br, 1), lambda i: (i, 0)),
        ],
        out_shape=[
            jax.ShapeDtypeStruct((N, 2), _f32),
            jax.ShapeDtypeStruct((N, 1), _f32),
        ],
    )(x, w, b)


def _delta_body(yc_ref, yr_ref, rl_ref, out_ref):
    out_ref[...] = jnp.abs(yc_ref[...] - yr_ref[...]) * rl_ref[0]


def _delta_call(yc, yr, rl):
    br = 80
    grid = N // br
    return pl.pallas_call(
        _delta_body,
        grid=(grid,),
        in_specs=[
            pl.BlockSpec((br, 1), lambda i: (i, 0)),
            pl.BlockSpec((1, N), lambda i: (0, 0)),
            pl.BlockSpec(memory_space=pltpu.SMEM),
        ],
        out_specs=pl.BlockSpec((br, N), lambda i: (i, 0)),
        out_shape=jax.ShapeDtypeStruct((N, N), _f32),
    )(yc, yr, rl)


def _selmat(a):
    return jnp.zeros((D, H), _f32).at[
        jnp.arange(D), jnp.arange(D) // HD].set(a.reshape(D))


def kernel(x, edge_index, h0, lamda, alpha, return_loss, cums, fc_w,
           attn_l, attn_r, gat_bias, w2, cls_w, cls_b):
    src = edge_index[0]
    dst = edge_index[1]
    lam = jnp.asarray(lamda).astype(_f32)
    al = jnp.asarray(alpha).astype(_f32)
    rl = jnp.asarray(return_loss).astype(_f32).reshape(1)

    idx = jnp.arange(D)
    qof = (idx % HD) * H + idx // HD
    pmat = jnp.zeros((D, D), _f32).at[idx, qof].set(1.0)
    pinv = jnp.zeros((D, D), _f32).at[qof, idx].set(1.0)
    rmat = jnp.zeros((HD, D), _f32).at[idx % H, idx].set(1.0)

    xc = x
    rec = [x]
    for l in range(1, L + 1):
        amat = _selmat(attn_l[l - 1])
        bmat = _selmat(attn_r[l - 1])
        ms = jnp.concatenate([pmat, amat, amat], axis=1)
        mt = jnp.concatenate([bmat, bmat], axis=1)
        ts, td = _prep_call(xc, fc_w[l - 1], ms, mt)
        accp, denp = _edge_call(src, dst, ts, td)
        theta = min(1.0, math.log(1.0 / l + 1.0))
        coefs = jnp.stack([
            theta * lam,
            (1.0 - theta) * lam * (1.0 - al),
            (1.0 - theta) * lam * al,
            jnp.zeros_like(lam),
        ]).reshape(1, 4)
        xc = _mix_call(accp, denp, h0, xc, w2[l - 1, :D], w2[l - 1, D:],
                       gat_bias[l - 1].reshape(1, D), rmat, pinv, coefs)
        rec.append(xc)

    logit, y = _cls_call(xc, cls_w, cls_b.reshape(1, 2))
    delta = _delta_call(y, y.reshape(1, N), rl)
    return (logit, delta, xc, tuple(rec))


# speedup vs baseline: 110.3514x; 2.8316x over previous
import functools
import math

import jax
import jax.numpy as jnp
from jax import lax
from jax.experimental import pallas as pl
from jax.experimental.pallas import tpu as pltpu
from jax.experimental.pallas import tpu_sc as plsc

N = 10000
E = 320000
D = 128
H = 8
HD = 16
L = 4

NC = 2
NS = 16
NW = NC * NS
EPW = E // NW
CH = 40
NCHUNK = EPW // CH
NP = NCHUNK // 2
RPS = N // NS
ZR = 25

TSW = D + HD
TDW = HD

_f32 = jnp.float32


def _prep_body(x_ref, w_ref, ms_ref, mt_ref, ts_ref, td_ref):
    f = jnp.dot(x_ref[...], w_ref[...], preferred_element_type=_f32)
    ts_ref[...] = jnp.dot(f, ms_ref[...], preferred_element_type=_f32)
    td_ref[...] = jnp.dot(f, mt_ref[...], preferred_element_type=_f32)


def _prep_call(x, w, ms, mt):
    br = 1000
    grid = N // br
    return pl.pallas_call(
        _prep_body,
        grid=(grid,),
        in_specs=[
            pl.BlockSpec((br, D), lambda i: (i, 0)),
            pl.BlockSpec((D, D), lambda i: (0, 0)),
            pl.BlockSpec((D, TSW), lambda i: (0, 0)),
            pl.BlockSpec((D, TDW), lambda i: (0, 0)),
        ],
        out_specs=[
            pl.BlockSpec((br, TSW), lambda i: (i, 0)),
            pl.BlockSpec((br, TDW), lambda i: (i, 0)),
        ],
        out_shape=[
            jax.ShapeDtypeStruct((N, TSW), _f32),
            jax.ShapeDtypeStruct((N, TDW), _f32),
        ],
    )(x, w, ms, mt)


def _compute_chunk(ts_v, td_v, wf_v):
    def edge_blk(ib, _):
        for t in range(4):
            i = ib * 4 + t
            e = ts_v[i, pl.ds(D, HD)] + td_v[i, :]
            e = jnp.maximum(e, 0.2 * e)
            e = jnp.minimum(e, 80.0)
            ee = jnp.exp(e)
            wf_v[i, pl.ds(D, HD)] = ee
            for m in range(H):
                wf_v[i, pl.ds(HD * m, HD)] = (
                    ee * ts_v[i, pl.ds(HD * m, HD)])
        return 0
    lax.fori_loop(0, CH // 4, edge_blk, 0)


def _copy_idx(src_ref, dst_ref):
    for o in (0, 16, CH - 16):
        dst_ref[pl.ds(o, 16)] = src_ref[pl.ds(o, 16)]


def _edge_body(src_h, dst_h, ts_h, td_h, acc_h,
               src_a, dst_a, sdst_a, ts_a, td_a, wf_a,
               src_b, dst_b, sdst_b, ts_b, td_b, wf_b,
               zacc_v, acc_s, ga, gb, sa, sb, ia, ib_):
    c = lax.axis_index("c")
    s = lax.axis_index("s")
    wid = c * NS + s
    zero = jnp.zeros((16,), _f32)

    def zfill_a(i, _):
        for k in range(TSW // 16):
            zacc_v[i, pl.ds(16 * k, 16)] = zero
        return 0
    lax.fori_loop(0, ZR, zfill_a, 0)
    row0 = s * RPS
    for j in range(RPS // ZR):
        pltpu.sync_copy(zacc_v, acc_s.at[pl.ds(row0 + j * ZR, ZR)])
    plsc.subcore_barrier()

    ebase = wid * EPW

    pltpu.sync_copy(src_h.at[pl.ds(ebase, CH)], src_a)
    pltpu.sync_copy(dst_h.at[pl.ds(ebase, CH)], dst_a)
    pltpu.async_copy(ts_h.at[src_a], ts_a, ga)
    pltpu.async_copy(td_h.at[dst_a], td_a, ga)
    pltpu.async_copy(src_h.at[pl.ds(ebase + CH, CH)], src_b, ib_)
    pltpu.async_copy(dst_h.at[pl.ds(ebase + CH, CH)], dst_b, ib_)

    def pair_body(k, _):
        pltpu.make_async_copy(src_h.at[pl.ds(ebase, CH)], src_b, ib_).wait()
        pltpu.make_async_copy(dst_h.at[pl.ds(ebase, CH)], dst_b, ib_).wait()
        pltpu.async_copy(ts_h.at[src_b], ts_b, gb)
        pltpu.async_copy(td_h.at[dst_b], td_b, gb)
        pltpu.make_async_copy(ts_h.at[src_a], ts_a, ga).wait()
        pltpu.make_async_copy(td_h.at[dst_a], td_a, ga).wait()

        @pl.when(k > 0)
        def _():
            pltpu.make_async_copy(wf_a, acc_s.at[sdst_a], sa).wait()
        _copy_idx(dst_a, sdst_a)

        @pl.when(k < NP - 1)
        def _():
            off2 = ebase + (2 * k + 2) * CH
            pltpu.async_copy(src_h.at[pl.ds(off2, CH)], src_a, ia)
            pltpu.async_copy(dst_h.at[pl.ds(off2, CH)], dst_a, ia)
        _compute_chunk(ts_a, td_a, wf_a)
        pltpu.async_copy(wf_a, acc_s.at[sdst_a], sa, add=True)

        @pl.when(k < NP - 1)
        def _():
            pltpu.make_async_copy(src_h.at[pl.ds(ebase, CH)], src_a,
                                  ia).wait()
            pltpu.make_async_copy(dst_h.at[pl.ds(ebase, CH)], dst_a,
                                  ia).wait()
            pltpu.async_copy(ts_h.at[src_a], ts_a, ga)
            pltpu.async_copy(td_h.at[dst_a], td_a, ga)

        pltpu.make_async_copy(ts_h.at[src_b], ts_b, gb).wait()
        pltpu.make_async_copy(td_h.at[dst_b], td_b, gb).wait()

        @pl.when(k > 0)
        def _():
            pltpu.make_async_copy(wf_b, acc_s.at[sdst_b], sb).wait()
        _copy_idx(dst_b, sdst_b)

        @pl.when(k < NP - 1)
        def _():
            off3 = ebase + (2 * k + 3) * CH
            pltpu.async_copy(src_h.at[pl.ds(off3, CH)], src_b, ib_)
            pltpu.async_copy(dst_h.at[pl.ds(off3, CH)], dst_b, ib_)
        _compute_chunk(ts_b, td_b, wf_b)
        pltpu.async_copy(wf_b, acc_s.at[sdst_b], sb, add=True)
        return 0
    lax.fori_loop(0, NP, pair_body, 0)
    pltpu.make_async_copy(wf_a, acc_s.at[sdst_a], sa).wait()
    pltpu.make_async_copy(wf_b, acc_s.at[sdst_b], sb).wait()
    plsc.subcore_barrier()

    for j in range(RPS // ZR):
        r = row0 + j * ZR
        pltpu.sync_copy(acc_s.at[pl.ds(r, ZR)], acc_h.at[c, pl.ds(r, ZR)])


_edge_call = functools.partial(
    pl.kernel,
    out_type=jax.ShapeDtypeStruct((NC, N, TSW), _f32),
    mesh=plsc.VectorSubcoreMesh(core_axis_name="c", subcore_axis_name="s"),
    scratch_types=[
        pltpu.VMEM((CH,), jnp.int32),
        pltpu.VMEM((CH,), jnp.int32),
        pltpu.VMEM((CH,), jnp.int32),
        pltpu.VMEM((CH, TSW), _f32),
        pltpu.VMEM((CH, TDW), _f32),
        pltpu.VMEM((CH, TSW), _f32),
        pltpu.VMEM((CH,), jnp.int32),
        pltpu.VMEM((CH,), jnp.int32),
        pltpu.VMEM((CH,), jnp.int32),
        pltpu.VMEM((CH, TSW), _f32),
        pltpu.VMEM((CH, TDW), _f32),
        pltpu.VMEM((CH, TSW), _f32),
        pltpu.VMEM((ZR, TSW), _f32),
        pltpu.VMEM_SHARED((N, TSW), _f32),
        pltpu.SemaphoreType.DMA,
        pltpu.SemaphoreType.DMA,
        pltpu.SemaphoreType.DMA,
        pltpu.SemaphoreType.DMA,
        pltpu.SemaphoreType.DMA,
        pltpu.SemaphoreType.DMA,
    ],
    compiler_params=pltpu.CompilerParams(use_tc_tiling_on_sc=False),
)(_edge_body)


def _mix_body(acc_ref, h0_ref, xp_ref, w2a_ref, w2b_ref,
              bias_ref, rmat_ref, pinv_ref, coef_ref, xn_ref):
    both = acc_ref[0] + acc_ref[1]
    accs = both[:, :D]
    dens = both[:, D:]
    den_exp = jnp.dot(dens, rmat_ref[...], preferred_element_type=_f32)
    gx_int = jnp.where(den_exp > 0.0, accs / den_exp, 0.0)
    gx = jnp.dot(gx_int, pinv_ref[...], preferred_element_type=_f32)
    gx = gx + bias_ref[...]
    h0 = h0_ref[...]
    mm = (jnp.dot(gx, w2a_ref[...], preferred_element_type=_f32)
          + jnp.dot(h0, w2b_ref[...], preferred_element_type=_f32))
    c1 = coef_ref[0, 0]
    c2 = coef_ref[0, 1]
    c3 = coef_ref[0, 2]
    xn_ref[...] = c1 * mm + c2 * gx + c3 * h0 + xp_ref[...]


def _mix_call(accp, h0, xp, w2a, w2b, bias, rmat, pinv, coefs):
    br = 1000
    grid = N // br
    return pl.pallas_call(
        _mix_body,
        grid=(grid,),
        in_specs=[
            pl.BlockSpec((NC, br, TSW), lambda i: (0, i, 0)),
            pl.BlockSpec((br, D), lambda i: (i, 0)),
            pl.BlockSpec((br, D), lambda i: (i, 0)),
            pl.BlockSpec((D, D), lambda i: (0, 0)),
            pl.BlockSpec((D, D), lambda i: (0, 0)),
            pl.BlockSpec((1, D), lambda i: (0, 0)),
            pl.BlockSpec((HD, D), lambda i: (0, 0)),
            pl.BlockSpec((D, D), lambda i: (0, 0)),
            pl.BlockSpec(memory_space=pltpu.SMEM),
        ],
        out_specs=pl.BlockSpec((br, D), lambda i: (i, 0)),
        out_shape=jax.ShapeDtypeStruct((N, D), _f32),
    )(accp, h0, xp, w2a, w2b, bias, rmat, pinv, coefs)


def _cls_body(x_ref, w_ref, b_ref, logit_ref, y_ref):
    lg = jnp.dot(x_ref[...], w_ref[...], preferred_element_type=_f32)
    lg = lg + b_ref[...]
    logit_ref[...] = lg
    y_ref[...] = jax.nn.sigmoid(lg[:, 1:2] - lg[:, 0:1])


def _cls_call(x, w, b):
    br = 1000
    grid = N // br
    return pl.pallas_call(
        _cls_body,
        grid=(grid,),
        in_specs=[
            pl.BlockSpec((br, D), lambda i: (i, 0)),
            pl.BlockSpec((D, 2), lambda i: (0, 0)),
            pl.BlockSpec((1, 2), lambda i: (0, 0)),
        ],
        out_specs=[
            pl.BlockSpec((br, 2), lambda i: (i, 0)),
            pl.BlockSpec((br, 1), lambda i: (i, 0)),
        ],
        out_shape=[
            jax.ShapeDtypeStruct((N, 2), _f32),
            jax.ShapeDtypeStruct((N, 1), _f32),
        ],
    )(x, w, b)


def _delta_body(yc_ref, yr_ref, rl_ref, out_ref):
    out_ref[...] = jnp.abs(yc_ref[...] - yr_ref[...]) * rl_ref[0]


def _delta_call(yc, yr, rl):
    br = 80
    grid = N // br
    return pl.pallas_call(
        _delta_body,
        grid=(grid,),
        in_specs=[
            pl.BlockSpec((br, 1), lambda i: (i, 0)),
            pl.BlockSpec((1, N), lambda i: (0, 0)),
            pl.BlockSpec(memory_space=pltpu.SMEM),
        ],
        out_specs=pl.BlockSpec((br, N), lambda i: (i, 0)),
        out_shape=jax.ShapeDtypeStruct((N, N), _f32),
    )(yc, yr, rl)


def _selmat(a):
    return jnp.zeros((D, H), _f32).at[
        jnp.arange(D), jnp.arange(D) // HD].set(a.reshape(D))


def kernel(x, edge_index, h0, lamda, alpha, return_loss, cums, fc_w,
           attn_l, attn_r, gat_bias, w2, cls_w, cls_b):
    src = edge_index[0]
    dst = edge_index[1]
    lam = jnp.asarray(lamda).astype(_f32)
    al = jnp.asarray(alpha).astype(_f32)
    rl = jnp.asarray(return_loss).astype(_f32).reshape(1)

    idx = jnp.arange(D)
    qof = (idx % HD) * H + idx // HD
    pmat = jnp.zeros((D, D), _f32).at[idx, qof].set(1.0)
    pinv = jnp.zeros((D, D), _f32).at[qof, idx].set(1.0)
    rmat = jnp.zeros((HD, D), _f32).at[idx % H, idx].set(1.0)

    xc = x
    rec = [x]
    for l in range(1, L + 1):
        amat = _selmat(attn_l[l - 1])
        bmat = _selmat(attn_r[l - 1])
        ms = jnp.concatenate([pmat, amat, amat], axis=1)
        mt = jnp.concatenate([bmat, bmat], axis=1)
        ts, td = _prep_call(xc, fc_w[l - 1], ms, mt)
        accp = _edge_call(src, dst, ts, td)
        theta = min(1.0, math.log(1.0 / l + 1.0))
        coefs = jnp.stack([
            theta * lam,
            (1.0 - theta) * lam * (1.0 - al),
            (1.0 - theta) * lam * al,
            jnp.zeros_like(lam),
        ]).reshape(1, 4)
        xc = _mix_call(accp, h0, xc, w2[l - 1, :D], w2[l - 1, D:],
                       gat_bias[l - 1].reshape(1, D), rmat, pinv, coefs)
        rec.append(xc)

    logit, y = _cls_call(xc, cls_w, cls_b.reshape(1, 2))
    delta = _delta_call(y, y.reshape(1, N), rl)
    return (logit, delta, xc, tuple(rec))


# speedup vs baseline: 111.8602x; 1.0137x over previous
import functools
import math

import jax
import jax.numpy as jnp
from jax import lax
from jax.experimental import pallas as pl
from jax.experimental.pallas import tpu as pltpu
from jax.experimental.pallas import tpu_sc as plsc

N = 10000
E = 320000
D = 128
H = 8
HD = 16
L = 4

NC = 2
NS = 16
NW = NC * NS
EPW = E // NW
CH = 40
NCHUNK = EPW // CH
NP = NCHUNK // 2
RPS = N // NS
ZR = 25

TSW = D + HD
TDW = HD

_f32 = jnp.float32


def _prep_body(x_ref, w_ref, ms_ref, mt_ref, ts_ref, td_ref):
    f = jnp.dot(x_ref[...], w_ref[...], preferred_element_type=_f32)
    ts_ref[...] = jnp.dot(f, ms_ref[...], preferred_element_type=_f32)
    td_ref[...] = jnp.dot(f, mt_ref[...], preferred_element_type=_f32)


def _prep_call(x, w, ms, mt):
    br = 1000
    grid = N // br
    return pl.pallas_call(
        _prep_body,
        grid=(grid,),
        in_specs=[
            pl.BlockSpec((br, D), lambda i: (i, 0)),
            pl.BlockSpec((D, D), lambda i: (0, 0)),
            pl.BlockSpec((D, TSW), lambda i: (0, 0)),
            pl.BlockSpec((D, TDW), lambda i: (0, 0)),
        ],
        out_specs=[
            pl.BlockSpec((br, TSW), lambda i: (i, 0)),
            pl.BlockSpec((br, TDW), lambda i: (i, 0)),
        ],
        out_shape=[
            jax.ShapeDtypeStruct((N, TSW), _f32),
            jax.ShapeDtypeStruct((N, TDW), _f32),
        ],
    )(x, w, ms, mt)


def _compute_chunk(ts_v, td_v, wf_v):
    def edge_blk(ib, _):
        for t in range(8):
            i = ib * 8 + t
            e = ts_v[i, pl.ds(D, HD)] + td_v[i, :]
            e = jnp.maximum(e, 0.2 * e)
            e = jnp.minimum(e, 80.0)
            ee = jnp.exp(e)
            wf_v[i, pl.ds(D, HD)] = ee
            for m in range(H):
                wf_v[i, pl.ds(HD * m, HD)] = (
                    ee * ts_v[i, pl.ds(HD * m, HD)])
        return 0
    lax.fori_loop(0, CH // 8, edge_blk, 0)


def _copy_idx(src_ref, dst_ref):
    for o in (0, 16, CH - 16):
        dst_ref[pl.ds(o, 16)] = src_ref[pl.ds(o, 16)]


def _edge_body(src_h, dst_h, ts_h, td_h, acc_h,
               src_a, dst_a, sdst_a, ts_a, td_a, wf_a,
               src_b, dst_b, sdst_b, ts_b, td_b, wf_b,
               zacc_v, acc_s, ga, gb, sa, sb, ia, ib_):
    c = lax.axis_index("c")
    s = lax.axis_index("s")
    wid = c * NS + s
    zero = jnp.zeros((16,), _f32)

    def zfill_a(i, _):
        for k in range(TSW // 16):
            zacc_v[i, pl.ds(16 * k, 16)] = zero
        return 0
    lax.fori_loop(0, ZR, zfill_a, 0)
    row0 = s * RPS
    for j in range(RPS // ZR):
        pltpu.sync_copy(zacc_v, acc_s.at[pl.ds(row0 + j * ZR, ZR)])
    plsc.subcore_barrier()

    ebase = wid * EPW

    pltpu.sync_copy(src_h.at[pl.ds(ebase, CH)], src_a)
    pltpu.sync_copy(dst_h.at[pl.ds(ebase, CH)], dst_a)
    pltpu.async_copy(ts_h.at[src_a], ts_a, ga)
    pltpu.async_copy(td_h.at[dst_a], td_a, ga)
    pltpu.async_copy(src_h.at[pl.ds(ebase + CH, CH)], src_b, ib_)
    pltpu.async_copy(dst_h.at[pl.ds(ebase + CH, CH)], dst_b, ib_)

    def pair_body(k, _):
        pltpu.make_async_copy(src_h.at[pl.ds(ebase, CH)], src_b, ib_).wait()
        pltpu.make_async_copy(dst_h.at[pl.ds(ebase, CH)], dst_b, ib_).wait()
        pltpu.async_copy(ts_h.at[src_b], ts_b, gb)
        pltpu.async_copy(td_h.at[dst_b], td_b, gb)
        pltpu.make_async_copy(ts_h.at[src_a], ts_a, ga).wait()
        pltpu.make_async_copy(td_h.at[dst_a], td_a, ga).wait()

        @pl.when(k > 0)
        def _():
            pltpu.make_async_copy(wf_a, acc_s.at[sdst_a], sa).wait()
        _copy_idx(dst_a, sdst_a)

        @pl.when(k < NP - 1)
        def _():
            off2 = ebase + (2 * k + 2) * CH
            pltpu.async_copy(src_h.at[pl.ds(off2, CH)], src_a, ia)
            pltpu.async_copy(dst_h.at[pl.ds(off2, CH)], dst_a, ia)
        _compute_chunk(ts_a, td_a, wf_a)
        pltpu.async_copy(wf_a, acc_s.at[sdst_a], sa, add=True)

        @pl.when(k < NP - 1)
        def _():
            pltpu.make_async_copy(src_h.at[pl.ds(ebase, CH)], src_a,
                                  ia).wait()
            pltpu.make_async_copy(dst_h.at[pl.ds(ebase, CH)], dst_a,
                                  ia).wait()
            pltpu.async_copy(ts_h.at[src_a], ts_a, ga)
            pltpu.async_copy(td_h.at[dst_a], td_a, ga)

        pltpu.make_async_copy(ts_h.at[src_b], ts_b, gb).wait()
        pltpu.make_async_copy(td_h.at[dst_b], td_b, gb).wait()

        @pl.when(k > 0)
        def _():
            pltpu.make_async_copy(wf_b, acc_s.at[sdst_b], sb).wait()
        _copy_idx(dst_b, sdst_b)

        @pl.when(k < NP - 1)
        def _():
            off3 = ebase + (2 * k + 3) * CH
            pltpu.async_copy(src_h.at[pl.ds(off3, CH)], src_b, ib_)
            pltpu.async_copy(dst_h.at[pl.ds(off3, CH)], dst_b, ib_)
        _compute_chunk(ts_b, td_b, wf_b)
        pltpu.async_copy(wf_b, acc_s.at[sdst_b], sb, add=True)
        return 0
    lax.fori_loop(0, NP, pair_body, 0)
    pltpu.make_async_copy(wf_a, acc_s.at[sdst_a], sa).wait()
    pltpu.make_async_copy(wf_b, acc_s.at[sdst_b], sb).wait()
    plsc.subcore_barrier()

    for j in range(RPS // ZR):
        r = row0 + j * ZR
        pltpu.sync_copy(acc_s.at[pl.ds(r, ZR)], acc_h.at[c, pl.ds(r, ZR)])


_edge_call = functools.partial(
    pl.kernel,
    out_type=jax.ShapeDtypeStruct((NC, N, TSW), _f32),
    mesh=plsc.VectorSubcoreMesh(core_axis_name="c", subcore_axis_name="s"),
    scratch_types=[
        pltpu.VMEM((CH,), jnp.int32),
        pltpu.VMEM((CH,), jnp.int32),
        pltpu.VMEM((CH,), jnp.int32),
        pltpu.VMEM((CH, TSW), _f32),
        pltpu.VMEM((CH, TDW), _f32),
        pltpu.VMEM((CH, TSW), _f32),
        pltpu.VMEM((CH,), jnp.int32),
        pltpu.VMEM((CH,), jnp.int32),
        pltpu.VMEM((CH,), jnp.int32),
        pltpu.VMEM((CH, TSW), _f32),
        pltpu.VMEM((CH, TDW), _f32),
        pltpu.VMEM((CH, TSW), _f32),
        pltpu.VMEM((ZR, TSW), _f32),
        pltpu.VMEM_SHARED((N, TSW), _f32),
        pltpu.SemaphoreType.DMA,
        pltpu.SemaphoreType.DMA,
        pltpu.SemaphoreType.DMA,
        pltpu.SemaphoreType.DMA,
        pltpu.SemaphoreType.DMA,
        pltpu.SemaphoreType.DMA,
    ],
    compiler_params=pltpu.CompilerParams(use_tc_tiling_on_sc=False),
)(_edge_body)


def _mix_body(acc_ref, h0_ref, xp_ref, w2a_ref, w2b_ref,
              bias_ref, rmat_ref, pinv_ref, coef_ref, xn_ref):
    both = acc_ref[0] + acc_ref[1]
    accs = both[:, :D]
    dens = both[:, D:]
    den_exp = jnp.dot(dens, rmat_ref[...], preferred_element_type=_f32)
    gx_int = jnp.where(den_exp > 0.0, accs / den_exp, 0.0)
    gx = jnp.dot(gx_int, pinv_ref[...], preferred_element_type=_f32)
    gx = gx + bias_ref[...]
    h0 = h0_ref[...]
    mm = (jnp.dot(gx, w2a_ref[...], preferred_element_type=_f32)
          + jnp.dot(h0, w2b_ref[...], preferred_element_type=_f32))
    c1 = coef_ref[0, 0]
    c2 = coef_ref[0, 1]
    c3 = coef_ref[0, 2]
    xn_ref[...] = c1 * mm + c2 * gx + c3 * h0 + xp_ref[...]


def _mix_call(accp, h0, xp, w2a, w2b, bias, rmat, pinv, coefs):
    br = 1000
    grid = N // br
    return pl.pallas_call(
        _mix_body,
        grid=(grid,),
        in_specs=[
            pl.BlockSpec((NC, br, TSW), lambda i: (0, i, 0)),
            pl.BlockSpec((br, D), lambda i: (i, 0)),
            pl.BlockSpec((br, D), lambda i: (i, 0)),
            pl.BlockSpec((D, D), lambda i: (0, 0)),
            pl.BlockSpec((D, D), lambda i: (0, 0)),
            pl.BlockSpec((1, D), lambda i: (0, 0)),
            pl.BlockSpec((HD, D), lambda i: (0, 0)),
            pl.BlockSpec((D, D), lambda i: (0, 0)),
            pl.BlockSpec(memory_space=pltpu.SMEM),
        ],
        out_specs=pl.BlockSpec((br, D), lambda i: (i, 0)),
        out_shape=jax.ShapeDtypeStruct((N, D), _f32),
    )(accp, h0, xp, w2a, w2b, bias, rmat, pinv, coefs)


def _cls_body(x_ref, w_ref, b_ref, logit_ref, y_ref):
    lg = jnp.dot(x_ref[...], w_ref[...], preferred_element_type=_f32)
    lg = lg + b_ref[...]
    logit_ref[...] = lg
    y_ref[...] = jax.nn.sigmoid(lg[:, 1:2] - lg[:, 0:1])


def _cls_call(x, w, b):
    br = 1000
    grid = N // br
    return pl.pallas_call(
        _cls_body,
        grid=(grid,),
        in_specs=[
            pl.BlockSpec((br, D), lambda i: (i, 0)),
            pl.BlockSpec((D, 2), lambda i: (0, 0)),
            pl.BlockSpec((1, 2), lambda i: (0, 0)),
        ],
        out_specs=[
            pl.BlockSpec((br, 2), lambda i: (i, 0)),
            pl.BlockSpec((br, 1), lambda i: (i, 0)),
        ],
        out_shape=[
            jax.ShapeDtypeStruct((N, 2), _f32),
            jax.ShapeDtypeStruct((N, 1), _f32),
        ],
    )(x, w, b)


def _delta_body(yc_ref, yr_ref, rl_ref, out_ref):
    out_ref[...] = jnp.abs(yc_ref[...] - yr_ref[...]) * rl_ref[0]


def _delta_call(yc, yr, rl):
    br = 200
    grid = N // br
    return pl.pallas_call(
        _delta_body,
        grid=(grid,),
        in_specs=[
            pl.BlockSpec((br, 1), lambda i: (i, 0)),
            pl.BlockSpec((1, N), lambda i: (0, 0)),
            pl.BlockSpec(memory_space=pltpu.SMEM),
        ],
        out_specs=pl.BlockSpec((br, N), lambda i: (i, 0)),
        out_shape=jax.ShapeDtypeStruct((N, N), _f32),
    )(yc, yr, rl)


def _selmat(a):
    return jnp.zeros((D, H), _f32).at[
        jnp.arange(D), jnp.arange(D) // HD].set(a.reshape(D))


def kernel(x, edge_index, h0, lamda, alpha, return_loss, cums, fc_w,
           attn_l, attn_r, gat_bias, w2, cls_w, cls_b):
    src = edge_index[0]
    dst = edge_index[1]
    lam = jnp.asarray(lamda).astype(_f32)
    al = jnp.asarray(alpha).astype(_f32)
    rl = jnp.asarray(return_loss).astype(_f32).reshape(1)

    idx = jnp.arange(D)
    qof = (idx % HD) * H + idx // HD
    pmat = jnp.zeros((D, D), _f32).at[idx, qof].set(1.0)
    pinv = jnp.zeros((D, D), _f32).at[qof, idx].set(1.0)
    rmat = jnp.zeros((HD, D), _f32).at[idx % H, idx].set(1.0)

    xc = x
    rec = [x]
    for l in range(1, L + 1):
        amat = _selmat(attn_l[l - 1])
        bmat = _selmat(attn_r[l - 1])
        ms = jnp.concatenate([pmat, amat, amat], axis=1)
        mt = jnp.concatenate([bmat, bmat], axis=1)
        ts, td = _prep_call(xc, fc_w[l - 1], ms, mt)
        accp = _edge_call(src, dst, ts, td)
        theta = min(1.0, math.log(1.0 / l + 1.0))
        coefs = jnp.stack([
            theta * lam,
            (1.0 - theta) * lam * (1.0 - al),
            (1.0 - theta) * lam * al,
            jnp.zeros_like(lam),
        ]).reshape(1, 4)
        xc = _mix_call(accp, h0, xc, w2[l - 1, :D], w2[l - 1, D:],
                       gat_bias[l - 1].reshape(1, D), rmat, pinv, coefs)
        rec.append(xc)

    logit, y = _cls_call(xc, cls_w, cls_b.reshape(1, 2))
    delta = _delta_call(y, y.reshape(1, N), rl)
    return (logit, delta, xc, tuple(rec))
